# Initial kernel scaffold; baseline (speedup 1.0000x reference)
import jax
import jax.numpy as jnp
from jax.experimental import pallas as pl


def kernel(sim_m, sim_g, edge_index, edge_type, label_edge, lin_m_w, lin_m_b, lin_g_w, lin_g_b, comp1, basis1, root1, bias1, comp2, basis2, root2, bias2):
    raise NotImplementedError("write your pallas kernel here")


# speedup vs baseline: 10.9491x; 10.9491x over previous
import functools

import jax
import jax.numpy as jnp
from jax import lax
from jax.experimental import pallas as pl
from jax.experimental.pallas import tpu as pltpu
from jax.experimental.pallas import tpu_sc as plsc

N_M = 5000
N_G = 5000
N = N_M + N_G
E = 320000
R = 8
D = 128
NCOL = (R + 1) * D
RN = R * N
B_EDGES = 8192

NC, NS, L = 2, 16, 16
NW = NC * NS
EPW = E // NW
EPS = E // NS
C = 80
WC = 200
PP = B_EDGES // NW
CD = 64


def _sc_mesh():
    return plsc.VectorSubcoreMesh(
        core_axis_name="c", subcore_axis_name="s",
        num_cores=NC, num_subcores=NS)


def _mm_body(x_ref, w_ref, b_ref, o_ref):
    o_ref[...] = jnp.dot(x_ref[...], w_ref[...],
                         preferred_element_type=jnp.float32) + b_ref[...]


def _matmul(x, w, b2d, bm):
    m, k = x.shape
    n = w.shape[1]
    return pl.pallas_call(
        _mm_body,
        grid=(m // bm,),
        in_specs=[pl.BlockSpec((bm, k), lambda i: (i, 0)),
                  pl.BlockSpec((k, n), lambda i: (0, 0)),
                  pl.BlockSpec((1, n), lambda i: (0, 0))],
        out_specs=pl.BlockSpec((bm, n), lambda i: (i, 0)),
        out_shape=jax.ShapeDtypeStruct((m, n), jnp.float32),
    )(x, w, b2d)


def _fused_body(a_ref, b_ref, y_ref, w_ref, bias_ref, o_ref):
    x = a_ref[0] + b_ref[0] + y_ref[...]
    o_ref[...] = jnp.dot(x, w_ref[...],
                         preferred_element_type=jnp.float32) + bias_ref[...]


def _fused_matmul(acc, hb_prev, w, b2d, bm):
    n = w.shape[1]
    return pl.pallas_call(
        _fused_body,
        grid=(N // bm,),
        in_specs=[pl.BlockSpec((1, bm, D), lambda i: (0, i, 0)),
                  pl.BlockSpec((1, bm, D), lambda i: (1, i, 0)),
                  pl.BlockSpec((bm, D), lambda i: (i, R)),
                  pl.BlockSpec((D, n), lambda i: (0, 0)),
                  pl.BlockSpec((1, n), lambda i: (0, 0))],
        out_specs=pl.BlockSpec((bm, n), lambda i: (i, 0)),
        out_shape=jax.ShapeDtypeStruct((N, n), jnp.float32),
    )(acc, acc, hb_prev, w, b2d)


def _add3_body(a_ref, b_ref, y_ref, o_ref):
    o_ref[...] = a_ref[0] + b_ref[0] + y_ref[...]


def _add3(acc, hb, bm):
    return pl.pallas_call(
        _add3_body,
        grid=(N // bm,),
        in_specs=[pl.BlockSpec((1, bm, D), lambda i: (0, i, 0)),
                  pl.BlockSpec((1, bm, D), lambda i: (1, i, 0)),
                  pl.BlockSpec((bm, D), lambda i: (i, R))],
        out_specs=pl.BlockSpec((bm, D), lambda i: (i, 0)),
        out_shape=jax.ShapeDtypeStruct((N, D), jnp.float32),
    )(acc, acc, hb)


def _prep_body(src_ref, dst_ref, et_ref, z8k_ref, ones_ref,
               hist0_ref, hist1_ref, norm_ref, gidx_ref,
               hist_s, tb, db, sb, cidxb, onesb, cb, nb, gb, zb, sem):
    cid = lax.axis_index("c")
    sid = lax.axis_index("s")
    wid = sid * NC + cid

    pltpu.sync_copy(ones_ref, onesb)

    @pl.when(sid < 10)
    def _():
        pltpu.sync_copy(z8k_ref, zb)
        pltpu.sync_copy(zb, hist_s.at[pl.ds(sid * 8000, 8000)])

    plsc.subcore_barrier()

    def chunk_a(ci, _):
        base = sid * EPS + ci * C
        pltpu.sync_copy(et_ref.at[pl.ds(base, C)], tb)
        pltpu.sync_copy(dst_ref.at[pl.ds(base, C)], db)

        def grp(j, _):
            t = tb[pl.ds(j * 16, 16)]
            dd = db[pl.ds(j * 16, 16)]
            cidxb[pl.ds(j * 16, 16)] = t * N + dd
            return 0
        lax.fori_loop(0, C // 16, grp, 0)
        pltpu.sync_copy(onesb, hist_s.at[cidxb], add=True)
        return 0
    lax.fori_loop(0, EPS // C, chunk_a, 0)

    plsc.subcore_barrier()

    @pl.when((cid == 0) & (sid < 10))
    def _():
        pltpu.sync_copy(hist_s.at[pl.ds(sid * 8000, 8000)], zb)
        pltpu.sync_copy(zb, hist0_ref.at[pl.ds(sid * 8000, 8000)])

    @pl.when((cid == 1) & (sid < 10))
    def _():
        pltpu.sync_copy(hist_s.at[pl.ds(sid * 8000, 8000)], zb)
        pltpu.sync_copy(zb, hist1_ref.at[pl.ds(sid * 8000, 8000)])

    plsc.subcore_barrier()

    def chunk_b(ci, _):
        base = wid * EPW + ci * C
        pltpu.sync_copy(et_ref.at[pl.ds(base, C)], tb)
        pltpu.sync_copy(src_ref.at[pl.ds(base, C)], sb)
        pltpu.sync_copy(dst_ref.at[pl.ds(base, C)], db)

        def grp(j, _):
            t = tb[pl.ds(j * 16, 16)]
            ss = sb[pl.ds(j * 16, 16)]
            dd = db[pl.ds(j * 16, 16)]
            cidxb[pl.ds(j * 16, 16)] = t * N + dd
            gb[pl.ds(j * 16, 16)] = ss * (R + 1) + t
            return 0
        lax.fori_loop(0, C // 16, grp, 0)

        @pl.when(cid == 0)
        def _():
            pltpu.async_copy(hist0_ref.at[cidxb], cb, sem).wait()

        @pl.when(cid == 1)
        def _():
            pltpu.async_copy(hist1_ref.at[cidxb], cb, sem).wait()

        def grp2(j, _):
            c16 = cb[pl.ds(j * 16, 16)]
            nb[pl.ds(j * 16, 16)] = 1.0 / jnp.maximum(c16, 1.0)
            return 0
        lax.fori_loop(0, C // 16, grp2, 0)
        pltpu.sync_copy(nb, norm_ref.at[pl.ds(base, C)])
        pltpu.sync_copy(gb, gidx_ref.at[pl.ds(base, C)])
        return 0
    lax.fori_loop(0, EPW // C, chunk_b, 0)


def _sc_prep(srcv, dstv, et, z8k, onesc):
    f = pl.kernel(
        _prep_body,
        out_type=[jax.ShapeDtypeStruct((RN,), jnp.float32),
                  jax.ShapeDtypeStruct((RN,), jnp.float32),
                  jax.ShapeDtypeStruct((E,), jnp.float32),
                  jax.ShapeDtypeStruct((E,), jnp.int32)],
        mesh=_sc_mesh(),
        scratch_types=[pltpu.VMEM_SHARED((RN,), jnp.float32),
                       pltpu.VMEM((C,), jnp.int32),
                       pltpu.VMEM((C,), jnp.int32),
                       pltpu.VMEM((C,), jnp.int32),
                       pltpu.VMEM((C,), jnp.int32),
                       pltpu.VMEM((C,), jnp.float32),
                       pltpu.VMEM((C,), jnp.float32),
                       pltpu.VMEM((C,), jnp.float32),
                       pltpu.VMEM((C,), jnp.int32),
                       pltpu.VMEM((8000,), jnp.float32),
                       pltpu.SemaphoreType.DMA],
        name="rgcn_sc_prep",
    )
    return f(srcv, dstv, et, z8k, onesc)


def _agg_body(hb_ref, dst_ref, gidx_ref, norm_ref, zwc_ref, acc_ref,
              acc_s, gb, db, nb, rows, zrows, sem):
    cid = lax.axis_index("c")
    sid = lax.axis_index("s")
    wid = sid * NC + cid

    pltpu.sync_copy(zwc_ref, zrows)

    def zcopy(kk, _):
        m = sid + kk * NS

        @pl.when(m < N // WC)
        def _():
            pltpu.sync_copy(zrows, acc_s.at[pl.ds(m * WC, WC)])
        return 0
    lax.fori_loop(0, (N // WC + NS - 1) // NS, zcopy, 0)

    plsc.subcore_barrier()

    def chunk(ci, _):
        base = wid * EPW + ci * C
        pltpu.sync_copy(gidx_ref.at[pl.ds(base, C)], gb)
        pltpu.sync_copy(dst_ref.at[pl.ds(base, C)], db)
        pltpu.sync_copy(norm_ref.at[pl.ds(base, C)], nb)
        pltpu.async_copy(hb_ref.at[gb], rows, sem).wait()

        def sgrp(g, _):
            n16 = nb[pl.ds(g * 16, 16)]
            for jj in range(16):
                nj = n16[jj]
                row = g * 16 + jj
                for k in range(D // 16):
                    rows[row, pl.ds(k * 16, 16)] = (
                        rows[row, pl.ds(k * 16, 16)] * nj)
            return 0
        lax.fori_loop(0, C // 16, sgrp, 0)
        pltpu.sync_copy(rows, acc_s.at[db], add=True)
        return 0
    lax.fori_loop(0, EPW // C, chunk, 0)

    plsc.subcore_barrier()

    def wb(kk, _):
        m = sid + kk * NS

        @pl.when(m < N // WC)
        def _():
            pltpu.sync_copy(acc_s.at[pl.ds(m * WC, WC)], zrows)

            @pl.when(cid == 0)
            def _():
                pltpu.sync_copy(zrows, acc_ref.at[0, pl.ds(m * WC, WC)])

            @pl.when(cid == 1)
            def _():
                pltpu.sync_copy(zrows, acc_ref.at[1, pl.ds(m * WC, WC)])
        return 0
    lax.fori_loop(0, (N // WC + NS - 1) // NS, wb, 0)


def _sc_agg(hb_flat, dstv, gidx, norm, zwc):
    f = pl.kernel(
        _agg_body,
        out_type=jax.ShapeDtypeStruct((2, N, D), jnp.float32),
        mesh=_sc_mesh(),
        scratch_types=[pltpu.VMEM_SHARED((N, D), jnp.float32),
                       pltpu.VMEM((C,), jnp.int32),
                       pltpu.VMEM((C,), jnp.int32),
                       pltpu.VMEM((C,), jnp.float32),
                       pltpu.VMEM((C, D), jnp.float32),
                       pltpu.VMEM((WC, D), jnp.float32),
                       pltpu.SemaphoreType.DMA],
        name="rgcn_sc_agg",
    )
    return f(hb_flat, dstv, gidx, norm, zwc)


def _dec_body(x2_ref, la_ref, lb_ref, s_ref, ai, bi, ra, rb, ob, sem):
    cid = lax.axis_index("c")
    sid = lax.axis_index("s")
    wid = sid * NC + cid

    def chunk(ci, _):
        base = wid * PP + ci * CD
        pltpu.sync_copy(la_ref.at[pl.ds(base, CD)], ai)
        pltpu.sync_copy(lb_ref.at[pl.ds(base, CD)], bi)

        def g(j, _):
            bi[pl.ds(j * 16, 16)] = bi[pl.ds(j * 16, 16)] + N_M
            return 0
        lax.fori_loop(0, CD // 16, g, 0)
        pltpu.async_copy(x2_ref.at[ai], ra, sem).wait()
        pltpu.async_copy(x2_ref.at[bi], rb, sem).wait()

        def dot_grp(g, _):
            lanes = lax.iota(jnp.int32, 16)
            r16 = jnp.zeros((16,), jnp.float32)
            for jj in range(16):
                row = g * 16 + jj
                acc = ra[row, pl.ds(0, 16)] * rb[row, pl.ds(0, 16)]
                for k in range(1, D // 16):
                    acc = acc + ra[row, pl.ds(k * 16, 16)] * rb[row, pl.ds(k * 16, 16)]
                s_jj = acc[0]
                for l in range(1, 16):
                    s_jj = s_jj + acc[l]
                r16 = jnp.where(lanes == jj, s_jj, r16)
            ob[pl.ds(g * 16, 16)] = r16
            return 0
        lax.fori_loop(0, CD // 16, dot_grp, 0)
        pltpu.sync_copy(ob, s_ref.at[pl.ds(base, CD)])
        return 0
    lax.fori_loop(0, PP // CD, chunk, 0)


def _sc_decoder(x2, la, lb):
    f = pl.kernel(
        _dec_body,
        out_type=jax.ShapeDtypeStruct((B_EDGES,), jnp.float32),
        mesh=_sc_mesh(),
        scratch_types=[pltpu.VMEM((CD,), jnp.int32),
                       pltpu.VMEM((CD,), jnp.int32),
                       pltpu.VMEM((CD, D), jnp.float32),
                       pltpu.VMEM((CD, D), jnp.float32),
                       pltpu.VMEM((CD,), jnp.float32),
                       pltpu.SemaphoreType.DMA],
        name="rgcn_sc_decoder",
    )
    return f(x2, la, lb)


def _wcat(comp, basis, root, bias):
    w = jnp.einsum('rb,bio->rio', comp, basis)
    wc = jnp.concatenate([w.transpose(1, 0, 2).reshape(D, R * D), root], axis=1)
    bc = jnp.concatenate([jnp.zeros((R * D,), jnp.float32), bias]).reshape(1, NCOL)
    return wc, bc


def kernel(sim_m, sim_g, edge_index, edge_type, label_edge,
           lin_m_w, lin_m_b, lin_g_w, lin_g_b,
           comp1, basis1, root1, bias1,
           comp2, basis2, root2, bias2):
    ei = edge_index.astype(jnp.int32)
    et = edge_type.astype(jnp.int32)
    srcv, dstv = ei[0], ei[1]
    le = label_edge.astype(jnp.int32)
    la, lb = le[0], le[1]

    wc1, bc1 = _wcat(comp1, basis1, root1, bias1)
    wc2, bc2 = _wcat(comp2, basis2, root2, bias2)

    xm0 = _matmul(sim_m, lin_m_w, lin_m_b.reshape(1, -1), 1000)
    xg0 = _matmul(sim_g, lin_g_w, lin_g_b.reshape(1, -1), 1000)
    x0 = jnp.concatenate([xm0, xg0], axis=0)

    z8k = jnp.zeros((8000,), jnp.float32)
    onesc = jnp.ones((C,), jnp.float32)
    zwc = jnp.zeros((WC, D), jnp.float32)
    _h0, _h1, norm, gidx = _sc_prep(srcv, dstv, et, z8k, onesc)

    hb1 = _matmul(x0, wc1, bc1, 1000)
    acc1 = _sc_agg(hb1.reshape((R + 1) * N, D), dstv, gidx, norm, zwc)
    hb2 = _fused_matmul(acc1, hb1, wc2, bc2, 1000)
    acc2 = _sc_agg(hb2.reshape((R + 1) * N, D), dstv, gidx, norm, zwc)
    x2 = _add3(acc2, hb2, 1000)
    return _sc_decoder(x2, la, lb)


# speedup vs baseline: 15.0566x; 1.3752x over previous
import functools

import jax
import jax.numpy as jnp
from jax import lax
from jax.experimental import pallas as pl
from jax.experimental.pallas import tpu as pltpu
from jax.experimental.pallas import tpu_sc as plsc

N_M = 5000
N_G = 5000
N = N_M + N_G
E = 320000
R = 8
D = 128
NCOL = (R + 1) * D
RN = R * N
B_EDGES = 8192

NC, NS, L = 2, 16, 16
NW = NC * NS
EPW = E // NW
EPS = E // NS
C = 80
BLK = 400
WC = 40
PP = B_EDGES // NW
CD = 64


def _sc_mesh():
    return plsc.VectorSubcoreMesh(
        core_axis_name="c", subcore_axis_name="s",
        num_cores=NC, num_subcores=NS)


def _mm_body(x_ref, w_ref, b_ref, o_ref):
    o_ref[...] = jnp.dot(x_ref[...], w_ref[...],
                         preferred_element_type=jnp.float32) + b_ref[...]


def _matmul(x, w, b2d, bm):
    m, k = x.shape
    n = w.shape[1]
    return pl.pallas_call(
        _mm_body,
        grid=(m // bm,),
        in_specs=[pl.BlockSpec((bm, k), lambda i: (i, 0)),
                  pl.BlockSpec((k, n), lambda i: (0, 0)),
                  pl.BlockSpec((1, n), lambda i: (0, 0))],
        out_specs=pl.BlockSpec((bm, n), lambda i: (i, 0)),
        out_shape=jax.ShapeDtypeStruct((m, n), jnp.float32),
    )(x, w, b2d)


def _fused_body(a_ref, b_ref, y_ref, w_ref, bias_ref, o_ref):
    x = a_ref[0] + b_ref[0] + y_ref[...]
    o_ref[...] = jnp.dot(x, w_ref[...],
                         preferred_element_type=jnp.float32) + bias_ref[...]


def _fused_matmul(acc, hb_prev, w, b2d, bm):
    n = w.shape[1]
    return pl.pallas_call(
        _fused_body,
        grid=(N // bm,),
        in_specs=[pl.BlockSpec((1, bm, D), lambda i: (0, i, 0)),
                  pl.BlockSpec((1, bm, D), lambda i: (1, i, 0)),
                  pl.BlockSpec((bm, D), lambda i: (i, R)),
                  pl.BlockSpec((D, n), lambda i: (0, 0)),
                  pl.BlockSpec((1, n), lambda i: (0, 0))],
        out_specs=pl.BlockSpec((bm, n), lambda i: (i, 0)),
        out_shape=jax.ShapeDtypeStruct((N, n), jnp.float32),
    )(acc, acc, hb_prev, w, b2d)


def _add3_body(a_ref, b_ref, y_ref, o_ref):
    o_ref[...] = a_ref[0] + b_ref[0] + y_ref[...]


def _add3(acc, hb, bm):
    return pl.pallas_call(
        _add3_body,
        grid=(N // bm,),
        in_specs=[pl.BlockSpec((1, bm, D), lambda i: (0, i, 0)),
                  pl.BlockSpec((1, bm, D), lambda i: (1, i, 0)),
                  pl.BlockSpec((bm, D), lambda i: (i, R))],
        out_specs=pl.BlockSpec((bm, D), lambda i: (i, 0)),
        out_shape=jax.ShapeDtypeStruct((N, D), jnp.float32),
    )(acc, acc, hb)


def _prep_body(src_ref, dst_ref, et_ref, z8k_ref, ones_ref,
               hist0_ref, hist1_ref, norm_ref, gidx_ref,
               hist_s, tb, db, sb, cidxb, onesb, cb, nb, gb, zb, sem):
    cid = lax.axis_index("c")
    sid = lax.axis_index("s")
    wid = sid * NC + cid

    pltpu.sync_copy(ones_ref, onesb)

    @pl.when(sid < 10)
    def _():
        pltpu.sync_copy(z8k_ref, zb)
        pltpu.sync_copy(zb, hist_s.at[pl.ds(sid * 8000, 8000)])

    plsc.subcore_barrier()

    def chunk_a(ci, _):
        base = sid * EPS + ci * C
        pltpu.sync_copy(et_ref.at[pl.ds(base, C)], tb)
        pltpu.sync_copy(dst_ref.at[pl.ds(base, C)], db)

        def grp(j, _):
            t = tb[pl.ds(j * 16, 16)]
            dd = db[pl.ds(j * 16, 16)]
            cidxb[pl.ds(j * 16, 16)] = t * N + dd
            return 0
        lax.fori_loop(0, C // 16, grp, 0)
        pltpu.sync_copy(onesb, hist_s.at[cidxb], add=True)
        return 0
    lax.fori_loop(0, EPS // C, chunk_a, 0)

    plsc.subcore_barrier()

    @pl.when((cid == 0) & (sid < 10))
    def _():
        pltpu.sync_copy(hist_s.at[pl.ds(sid * 8000, 8000)], zb)
        pltpu.sync_copy(zb, hist0_ref.at[pl.ds(sid * 8000, 8000)])

    @pl.when((cid == 1) & (sid < 10))
    def _():
        pltpu.sync_copy(hist_s.at[pl.ds(sid * 8000, 8000)], zb)
        pltpu.sync_copy(zb, hist1_ref.at[pl.ds(sid * 8000, 8000)])

    plsc.subcore_barrier()

    def chunk_b(ci, _):
        base = wid * EPW + ci * C
        pltpu.sync_copy(et_ref.at[pl.ds(base, C)], tb)
        pltpu.sync_copy(src_ref.at[pl.ds(base, C)], sb)
        pltpu.sync_copy(dst_ref.at[pl.ds(base, C)], db)

        def grp(j, _):
            t = tb[pl.ds(j * 16, 16)]
            ss = sb[pl.ds(j * 16, 16)]
            dd = db[pl.ds(j * 16, 16)]
            cidxb[pl.ds(j * 16, 16)] = t * N + dd
            gb[pl.ds(j * 16, 16)] = ss * (R + 1) + t
            return 0
        lax.fori_loop(0, C // 16, grp, 0)

        @pl.when(cid == 0)
        def _():
            pltpu.async_copy(hist0_ref.at[cidxb], cb, sem).wait()

        @pl.when(cid == 1)
        def _():
            pltpu.async_copy(hist1_ref.at[cidxb], cb, sem).wait()

        def grp2(j, _):
            c16 = cb[pl.ds(j * 16, 16)]
            nb[pl.ds(j * 16, 16)] = 1.0 / jnp.maximum(c16, 1.0)
            return 0
        lax.fori_loop(0, C // 16, grp2, 0)
        pltpu.sync_copy(nb, norm_ref.at[pl.ds(base, C)])
        pltpu.sync_copy(gb, gidx_ref.at[pl.ds(base, C)])
        return 0
    lax.fori_loop(0, EPW // C, chunk_b, 0)


def _sc_prep(srcv, dstv, et, z8k, onesc):
    f = pl.kernel(
        _prep_body,
        out_type=[jax.ShapeDtypeStruct((RN,), jnp.float32),
                  jax.ShapeDtypeStruct((RN,), jnp.float32),
                  jax.ShapeDtypeStruct((E,), jnp.float32),
                  jax.ShapeDtypeStruct((E,), jnp.int32)],
        mesh=_sc_mesh(),
        scratch_types=[pltpu.VMEM_SHARED((RN,), jnp.float32),
                       pltpu.VMEM((C,), jnp.int32),
                       pltpu.VMEM((C,), jnp.int32),
                       pltpu.VMEM((C,), jnp.int32),
                       pltpu.VMEM((C,), jnp.int32),
                       pltpu.VMEM((C,), jnp.float32),
                       pltpu.VMEM((C,), jnp.float32),
                       pltpu.VMEM((C,), jnp.float32),
                       pltpu.VMEM((C,), jnp.int32),
                       pltpu.VMEM((8000,), jnp.float32),
                       pltpu.SemaphoreType.DMA],
        name="rgcn_sc_prep",
    )
    return f(srcv, dstv, et, z8k, onesc)


def _agg_body(hb_ref, dst_ref, gidx_ref, norm_ref, zwc_ref, acc_ref,
              acc_s, gblk, dblk, nblk, rows0, rows1, gb0, gb1, db0, db1,
              zrows, sem0, sem1):
    cid = lax.axis_index("c")
    sid = lax.axis_index("s")
    wid = sid * NC + cid

    pltpu.sync_copy(zwc_ref, zrows)

    def zcopy(kk, _):
        m = sid + kk * NS

        @pl.when(m < N // WC)
        def _():
            pltpu.sync_copy(zrows, acc_s.at[pl.ds(m * WC, WC)])
        return 0
    lax.fori_loop(0, (N // WC + NS - 1) // NS, zcopy, 0)

    plsc.subcore_barrier()

    def fill_idx(gbuf, dbuf, c):
        def g16(j, _):
            gbuf[pl.ds(j * 16, 16)] = gblk[pl.ds(c * C + j * 16, 16)]
            dbuf[pl.ds(j * 16, 16)] = dblk[pl.ds(c * C + j * 16, 16)]
            return 0
        lax.fori_loop(0, C // 16, g16, 0)

    def scale(rows, c):
        def sgrp(g, _):
            n16 = nblk[pl.ds(c * C + g * 16, 16)]
            for jj in range(16):
                nj = n16[jj]
                row = g * 16 + jj
                for k in range(D // 16):
                    rows[row, pl.ds(k * 16, 16)] = (
                        rows[row, pl.ds(k * 16, 16)] * nj)
            return 0
        lax.fori_loop(0, C // 16, sgrp, 0)

    CPB = BLK // C

    def block(bi, _):
        bbase = wid * EPW + bi * BLK
        pltpu.sync_copy(gidx_ref.at[pl.ds(bbase, BLK)], gblk)
        pltpu.sync_copy(dst_ref.at[pl.ds(bbase, BLK)], dblk)
        pltpu.sync_copy(norm_ref.at[pl.ds(bbase, BLK)], nblk)

        fill_idx(gb0, db0, 0)
        pltpu.async_copy(hb_ref.at[gb0], rows0, sem0)

        def pair(p, _):
            c1 = 2 * p + 1
            c2 = 2 * p + 2
            fill_idx(gb1, db1, c1)
            pltpu.async_copy(hb_ref.at[gb1], rows1, sem1)
            pltpu.make_async_copy(hb_ref.at[gb0], rows0, sem0).wait()
            scale(rows0, 2 * p)
            pltpu.sync_copy(rows0, acc_s.at[db0], add=True)
            fill_idx(gb0, db0, c2)
            pltpu.async_copy(hb_ref.at[gb0], rows0, sem0)
            pltpu.make_async_copy(hb_ref.at[gb1], rows1, sem1).wait()
            scale(rows1, c1)
            pltpu.sync_copy(rows1, acc_s.at[db1], add=True)
            return 0
        lax.fori_loop(0, (CPB - 1) // 2, pair, 0)

        pltpu.make_async_copy(hb_ref.at[gb0], rows0, sem0).wait()
        scale(rows0, CPB - 1)
        pltpu.sync_copy(rows0, acc_s.at[db0], add=True)
        return 0
    lax.fori_loop(0, EPW // BLK, block, 0)

    plsc.subcore_barrier()

    def wb(kk, _):
        m = sid + kk * NS

        @pl.when(m < N // WC)
        def _():
            pltpu.sync_copy(acc_s.at[pl.ds(m * WC, WC)], zrows)

            @pl.when(cid == 0)
            def _():
                pltpu.sync_copy(zrows, acc_ref.at[0, pl.ds(m * WC, WC)])

            @pl.when(cid == 1)
            def _():
                pltpu.sync_copy(zrows, acc_ref.at[1, pl.ds(m * WC, WC)])
        return 0
    lax.fori_loop(0, (N // WC + NS - 1) // NS, wb, 0)


def _sc_agg(hb_flat, dstv, gidx, norm, zwc):
    f = pl.kernel(
        _agg_body,
        out_type=jax.ShapeDtypeStruct((2, N, D), jnp.float32),
        mesh=_sc_mesh(),
        scratch_types=[pltpu.VMEM_SHARED((N, D), jnp.float32),
                       pltpu.VMEM((BLK,), jnp.int32),
                       pltpu.VMEM((BLK,), jnp.int32),
                       pltpu.VMEM((BLK,), jnp.float32),
                       pltpu.VMEM((C, D), jnp.float32),
                       pltpu.VMEM((C, D), jnp.float32),
                       pltpu.VMEM((C,), jnp.int32),
                       pltpu.VMEM((C,), jnp.int32),
                       pltpu.VMEM((C,), jnp.int32),
                       pltpu.VMEM((C,), jnp.int32),
                       pltpu.VMEM((WC, D), jnp.float32),
                       pltpu.SemaphoreType.DMA,
                       pltpu.SemaphoreType.DMA],
        name="rgcn_sc_agg",
    )
    return f(hb_flat, dstv, gidx, norm, zwc)


def _dec_body(x2_ref, la_ref, lb_ref, s_ref, ai, bi, ra, rb, ob, sem):
    cid = lax.axis_index("c")
    sid = lax.axis_index("s")
    wid = sid * NC + cid

    def chunk(ci, _):
        base = wid * PP + ci * CD
        pltpu.sync_copy(la_ref.at[pl.ds(base, CD)], ai)
        pltpu.sync_copy(lb_ref.at[pl.ds(base, CD)], bi)

        def g(j, _):
            bi[pl.ds(j * 16, 16)] = bi[pl.ds(j * 16, 16)] + N_M
            return 0
        lax.fori_loop(0, CD // 16, g, 0)
        pltpu.async_copy(x2_ref.at[ai], ra, sem).wait()
        pltpu.async_copy(x2_ref.at[bi], rb, sem).wait()

        def dot_grp(g, _):
            lanes = lax.iota(jnp.int32, 16)
            r16 = jnp.zeros((16,), jnp.float32)
            for jj in range(16):
                row = g * 16 + jj
                acc = ra[row, pl.ds(0, 16)] * rb[row, pl.ds(0, 16)]
                for k in range(1, D // 16):
                    acc = acc + ra[row, pl.ds(k * 16, 16)] * rb[row, pl.ds(k * 16, 16)]
                s_jj = acc[0]
                for l in range(1, 16):
                    s_jj = s_jj + acc[l]
                r16 = jnp.where(lanes == jj, s_jj, r16)
            ob[pl.ds(g * 16, 16)] = r16
            return 0
        lax.fori_loop(0, CD // 16, dot_grp, 0)
        pltpu.sync_copy(ob, s_ref.at[pl.ds(base, CD)])
        return 0
    lax.fori_loop(0, PP // CD, chunk, 0)


def _sc_decoder(x2, la, lb):
    f = pl.kernel(
        _dec_body,
        out_type=jax.ShapeDtypeStruct((B_EDGES,), jnp.float32),
        mesh=_sc_mesh(),
        scratch_types=[pltpu.VMEM((CD,), jnp.int32),
                       pltpu.VMEM((CD,), jnp.int32),
                       pltpu.VMEM((CD, D), jnp.float32),
                       pltpu.VMEM((CD, D), jnp.float32),
                       pltpu.VMEM((CD,), jnp.float32),
                       pltpu.SemaphoreType.DMA],
        name="rgcn_sc_decoder",
    )
    return f(x2, la, lb)


def _wcat(comp, basis, root, bias):
    w = jnp.einsum('rb,bio->rio', comp, basis)
    wc = jnp.concatenate([w.transpose(1, 0, 2).reshape(D, R * D), root], axis=1)
    bc = jnp.concatenate([jnp.zeros((R * D,), jnp.float32), bias]).reshape(1, NCOL)
    return wc, bc


def kernel(sim_m, sim_g, edge_index, edge_type, label_edge,
           lin_m_w, lin_m_b, lin_g_w, lin_g_b,
           comp1, basis1, root1, bias1,
           comp2, basis2, root2, bias2):
    ei = edge_index.astype(jnp.int32)
    et = edge_type.astype(jnp.int32)
    srcv, dstv = ei[0], ei[1]
    le = label_edge.astype(jnp.int32)
    la, lb = le[0], le[1]

    wc1, bc1 = _wcat(comp1, basis1, root1, bias1)
    wc2, bc2 = _wcat(comp2, basis2, root2, bias2)

    xm0 = _matmul(sim_m, lin_m_w, lin_m_b.reshape(1, -1), 1000)
    xg0 = _matmul(sim_g, lin_g_w, lin_g_b.reshape(1, -1), 1000)
    x0 = jnp.concatenate([xm0, xg0], axis=0)

    z8k = jnp.zeros((8000,), jnp.float32)
    onesc = jnp.ones((C,), jnp.float32)
    zwc = jnp.zeros((WC, D), jnp.float32)
    _h0, _h1, norm, gidx = _sc_prep(srcv, dstv, et, z8k, onesc)

    hb1 = _matmul(x0, wc1, bc1, 1000)
    acc1 = _sc_agg(hb1.reshape((R + 1) * N, D), dstv, gidx, norm, zwc)
    hb2 = _fused_matmul(acc1, hb1, wc2, bc2, 1000)
    acc2 = _sc_agg(hb2.reshape((R + 1) * N, D), dstv, gidx, norm, zwc)
    x2 = _add3(acc2, hb2, 1000)
    return _sc_decoder(x2, la, lb)


# speedup vs baseline: 21.5990x; 1.4345x over previous
import functools

import jax
import jax.numpy as jnp
from jax import lax
from jax.experimental import pallas as pl
from jax.experimental.pallas import tpu as pltpu
from jax.experimental.pallas import tpu_sc as plsc

N_M = 5000
N_G = 5000
N = N_M + N_G
E = 320000
R = 8
D = 128
NCOL = (R + 1) * D
RN = R * N
B_EDGES = 8192

NC, NS, L = 2, 16, 16
NW = NC * NS
EPW = E // NW
EPS = E // NS
C = 80
BLK = 400
WC = 40
PP = B_EDGES // NW
CD = 64


def _sc_mesh():
    return plsc.VectorSubcoreMesh(
        core_axis_name="c", subcore_axis_name="s",
        num_cores=NC, num_subcores=NS)


def _mm_body(x_ref, w_ref, b_ref, o_ref):
    o_ref[...] = jnp.dot(x_ref[...], w_ref[...],
                         preferred_element_type=jnp.float32) + b_ref[...]


def _matmul(x, w, b2d, bm):
    m, k = x.shape
    n = w.shape[1]
    return pl.pallas_call(
        _mm_body,
        grid=(m // bm,),
        in_specs=[pl.BlockSpec((bm, k), lambda i: (i, 0)),
                  pl.BlockSpec((k, n), lambda i: (0, 0)),
                  pl.BlockSpec((1, n), lambda i: (0, 0))],
        out_specs=pl.BlockSpec((bm, n), lambda i: (i, 0)),
        out_shape=jax.ShapeDtypeStruct((m, n), jnp.float32),
    )(x, w, b2d)


def _fused_body(a_ref, b_ref, y_ref, w_ref, bias_ref, o_ref):
    x = a_ref[0] + b_ref[0] + y_ref[...]
    o_ref[...] = jnp.dot(x, w_ref[...],
                         preferred_element_type=jnp.float32) + bias_ref[...]


def _fused_matmul(acc, hb_prev, w, b2d, bm):
    n = w.shape[1]
    return pl.pallas_call(
        _fused_body,
        grid=(N // bm,),
        in_specs=[pl.BlockSpec((1, bm, D), lambda i: (0, i, 0)),
                  pl.BlockSpec((1, bm, D), lambda i: (1, i, 0)),
                  pl.BlockSpec((bm, D), lambda i: (i, R)),
                  pl.BlockSpec((D, n), lambda i: (0, 0)),
                  pl.BlockSpec((1, n), lambda i: (0, 0))],
        out_specs=pl.BlockSpec((bm, n), lambda i: (i, 0)),
        out_shape=jax.ShapeDtypeStruct((N, n), jnp.float32),
    )(acc, acc, hb_prev, w, b2d)


def _add3_body(a_ref, b_ref, y_ref, o_ref):
    o_ref[...] = a_ref[0] + b_ref[0] + y_ref[...]


def _add3(acc, hb, bm):
    return pl.pallas_call(
        _add3_body,
        grid=(N // bm,),
        in_specs=[pl.BlockSpec((1, bm, D), lambda i: (0, i, 0)),
                  pl.BlockSpec((1, bm, D), lambda i: (1, i, 0)),
                  pl.BlockSpec((bm, D), lambda i: (i, R))],
        out_specs=pl.BlockSpec((bm, D), lambda i: (i, 0)),
        out_shape=jax.ShapeDtypeStruct((N, D), jnp.float32),
    )(acc, acc, hb)


def _prep_body(src_ref, dst_ref, et_ref, z8k_ref, ones_ref,
               hist0_ref, hist1_ref, norm_ref, gidx_ref,
               hist_s, tblk, dblk, sblk, gxblk, nmblk,
               cidx0, cidx1, cb0, cb1, onesb, zb, sem0, sem1):
    cid = lax.axis_index("c")
    sid = lax.axis_index("s")
    wid = sid * NC + cid
    CPB = BLK // C

    pltpu.sync_copy(ones_ref, onesb)

    @pl.when(sid < 10)
    def _():
        pltpu.sync_copy(z8k_ref, zb)
        pltpu.sync_copy(zb, hist_s.at[pl.ds(sid * 8000, 8000)])

    plsc.subcore_barrier()

    def cidx_fill(buf, c):
        def g16(j, _):
            t = tblk[pl.ds(c * C + j * 16, 16)]
            dd = dblk[pl.ds(c * C + j * 16, 16)]
            buf[pl.ds(j * 16, 16)] = t * N + dd
            return 0
        lax.fori_loop(0, C // 16, g16, 0)

    def block_a(bi, _):
        bbase = sid * EPS + bi * BLK
        pltpu.sync_copy(et_ref.at[pl.ds(bbase, BLK)], tblk)
        pltpu.sync_copy(dst_ref.at[pl.ds(bbase, BLK)], dblk)
        cidx_fill(cidx0, 0)
        pltpu.async_copy(onesb, hist_s.at[cidx0], sem0, add=True)

        def pair(p, _):
            c1 = 2 * p + 1
            c2 = 2 * p + 2
            cidx_fill(cidx1, c1)
            pltpu.async_copy(onesb, hist_s.at[cidx1], sem1, add=True)
            pltpu.make_async_copy(onesb, hist_s.at[cidx0], sem0).wait()
            cidx_fill(cidx0, c2)
            pltpu.async_copy(onesb, hist_s.at[cidx0], sem0, add=True)
            pltpu.make_async_copy(onesb, hist_s.at[cidx1], sem1).wait()
            return 0
        lax.fori_loop(0, (CPB - 1) // 2, pair, 0)
        pltpu.make_async_copy(onesb, hist_s.at[cidx0], sem0).wait()
        return 0
    lax.fori_loop(0, EPS // BLK, block_a, 0)

    plsc.subcore_barrier()

    @pl.when((cid == 0) & (sid < 10))
    def _():
        pltpu.sync_copy(hist_s.at[pl.ds(sid * 8000, 8000)], zb)
        pltpu.sync_copy(zb, hist0_ref.at[pl.ds(sid * 8000, 8000)])

    @pl.when((cid == 1) & (sid < 10))
    def _():
        pltpu.sync_copy(hist_s.at[pl.ds(sid * 8000, 8000)], zb)
        pltpu.sync_copy(zb, hist1_ref.at[pl.ds(sid * 8000, 8000)])

    plsc.subcore_barrier()

    def start_cnt(ibuf, cbuf, s):
        @pl.when(cid == 0)
        def _():
            pltpu.async_copy(hist0_ref.at[ibuf], cbuf, s)

        @pl.when(cid == 1)
        def _():
            pltpu.async_copy(hist1_ref.at[ibuf], cbuf, s)

    def wait_cnt(ibuf, cbuf, s):
        @pl.when(cid == 0)
        def _():
            pltpu.make_async_copy(hist0_ref.at[ibuf], cbuf, s).wait()

        @pl.when(cid == 1)
        def _():
            pltpu.make_async_copy(hist1_ref.at[ibuf], cbuf, s).wait()

    def norm_fill(c, cbuf):
        def g16(j, _):
            c16 = cbuf[pl.ds(j * 16, 16)]
            nmblk[pl.ds(c * C + j * 16, 16)] = 1.0 / jnp.maximum(c16, 1.0)
            return 0
        lax.fori_loop(0, C // 16, g16, 0)

    def block_b(bi, _):
        bbase = wid * EPW + bi * BLK
        pltpu.sync_copy(et_ref.at[pl.ds(bbase, BLK)], tblk)
        pltpu.sync_copy(src_ref.at[pl.ds(bbase, BLK)], sblk)
        pltpu.sync_copy(dst_ref.at[pl.ds(bbase, BLK)], dblk)

        def gx(j, _):
            t = tblk[pl.ds(j * 16, 16)]
            ss = sblk[pl.ds(j * 16, 16)]
            gxblk[pl.ds(j * 16, 16)] = ss * (R + 1) + t
            return 0
        lax.fori_loop(0, BLK // 16, gx, 0)
        pltpu.sync_copy(gxblk, gidx_ref.at[pl.ds(bbase, BLK)])

        cidx_fill(cidx0, 0)
        start_cnt(cidx0, cb0, sem0)

        def pairb(p, _):
            c1 = 2 * p + 1
            c2 = 2 * p + 2
            cidx_fill(cidx1, c1)
            start_cnt(cidx1, cb1, sem1)
            wait_cnt(cidx0, cb0, sem0)
            norm_fill(2 * p, cb0)
            cidx_fill(cidx0, c2)
            start_cnt(cidx0, cb0, sem0)
            wait_cnt(cidx1, cb1, sem1)
            norm_fill(c1, cb1)
            return 0
        lax.fori_loop(0, (CPB - 1) // 2, pairb, 0)
        wait_cnt(cidx0, cb0, sem0)
        norm_fill(CPB - 1, cb0)
        pltpu.sync_copy(nmblk, norm_ref.at[pl.ds(bbase, BLK)])
        return 0
    lax.fori_loop(0, EPW // BLK, block_b, 0)


def _sc_prep(srcv, dstv, et, z8k, onesc):
    f = pl.kernel(
        _prep_body,
        out_type=[jax.ShapeDtypeStruct((RN,), jnp.float32),
                  jax.ShapeDtypeStruct((RN,), jnp.float32),
                  jax.ShapeDtypeStruct((E,), jnp.float32),
                  jax.ShapeDtypeStruct((E,), jnp.int32)],
        mesh=_sc_mesh(),
        scratch_types=[pltpu.VMEM_SHARED((RN,), jnp.float32),
                       pltpu.VMEM((BLK,), jnp.int32),
                       pltpu.VMEM((BLK,), jnp.int32),
                       pltpu.VMEM((BLK,), jnp.int32),
                       pltpu.VMEM((BLK,), jnp.int32),
                       pltpu.VMEM((BLK,), jnp.float32),
                       pltpu.VMEM((C,), jnp.int32),
                       pltpu.VMEM((C,), jnp.int32),
                       pltpu.VMEM((C,), jnp.float32),
                       pltpu.VMEM((C,), jnp.float32),
                       pltpu.VMEM((C,), jnp.float32),
                       pltpu.VMEM((8000,), jnp.float32),
                       pltpu.SemaphoreType.DMA,
                       pltpu.SemaphoreType.DMA],
        name="rgcn_sc_prep",
    )
    return f(srcv, dstv, et, z8k, onesc)


def _agg_body(hb_ref, dst_ref, gidx_ref, norm_ref, zwc_ref, acc_ref,
              acc_s, gblk, dblk, nblk, rows0, rows1, gb0, gb1, db0, db1,
              zrows, sem0, sem1):
    cid = lax.axis_index("c")
    sid = lax.axis_index("s")
    wid = sid * NC + cid

    pltpu.sync_copy(zwc_ref, zrows)

    def zcopy(kk, _):
        m = sid + kk * NS

        @pl.when(m < N // WC)
        def _():
            pltpu.sync_copy(zrows, acc_s.at[pl.ds(m * WC, WC)])
        return 0
    lax.fori_loop(0, (N // WC + NS - 1) // NS, zcopy, 0)

    plsc.subcore_barrier()

    def fill_idx(gbuf, dbuf, c):
        def g16(j, _):
            gbuf[pl.ds(j * 16, 16)] = gblk[pl.ds(c * C + j * 16, 16)]
            dbuf[pl.ds(j * 16, 16)] = dblk[pl.ds(c * C + j * 16, 16)]
            return 0
        lax.fori_loop(0, C // 16, g16, 0)

    def scale(rows, c):
        def sgrp(g, _):
            n16 = nblk[pl.ds(c * C + g * 16, 16)]
            for jj in range(16):
                nj = n16[jj]
                row = g * 16 + jj
                for k in range(D // 16):
                    rows[row, pl.ds(k * 16, 16)] = (
                        rows[row, pl.ds(k * 16, 16)] * nj)
            return 0
        lax.fori_loop(0, C // 16, sgrp, 0)

    CPB = BLK // C

    def block(bi, _):
        bbase = wid * EPW + bi * BLK
        pltpu.sync_copy(gidx_ref.at[pl.ds(bbase, BLK)], gblk)
        pltpu.sync_copy(dst_ref.at[pl.ds(bbase, BLK)], dblk)
        pltpu.sync_copy(norm_ref.at[pl.ds(bbase, BLK)], nblk)

        fill_idx(gb0, db0, 0)
        pltpu.async_copy(hb_ref.at[gb0], rows0, sem0)

        def pair(p, _):
            c1 = 2 * p + 1
            c2 = 2 * p + 2
            fill_idx(gb1, db1, c1)
            pltpu.async_copy(hb_ref.at[gb1], rows1, sem1)
            pltpu.make_async_copy(hb_ref.at[gb0], rows0, sem0).wait()
            scale(rows0, 2 * p)
            pltpu.sync_copy(rows0, acc_s.at[db0], add=True)
            fill_idx(gb0, db0, c2)
            pltpu.async_copy(hb_ref.at[gb0], rows0, sem0)
            pltpu.make_async_copy(hb_ref.at[gb1], rows1, sem1).wait()
            scale(rows1, c1)
            pltpu.sync_copy(rows1, acc_s.at[db1], add=True)
            return 0
        lax.fori_loop(0, (CPB - 1) // 2, pair, 0)

        pltpu.make_async_copy(hb_ref.at[gb0], rows0, sem0).wait()
        scale(rows0, CPB - 1)
        pltpu.sync_copy(rows0, acc_s.at[db0], add=True)
        return 0
    lax.fori_loop(0, EPW // BLK, block, 0)

    plsc.subcore_barrier()

    def wb(kk, _):
        m = sid + kk * NS

        @pl.when(m < N // WC)
        def _():
            pltpu.sync_copy(acc_s.at[pl.ds(m * WC, WC)], zrows)

            @pl.when(cid == 0)
            def _():
                pltpu.sync_copy(zrows, acc_ref.at[0, pl.ds(m * WC, WC)])

            @pl.when(cid == 1)
            def _():
                pltpu.sync_copy(zrows, acc_ref.at[1, pl.ds(m * WC, WC)])
        return 0
    lax.fori_loop(0, (N // WC + NS - 1) // NS, wb, 0)


def _sc_agg(hb_flat, dstv, gidx, norm, zwc):
    f = pl.kernel(
        _agg_body,
        out_type=jax.ShapeDtypeStruct((2, N, D), jnp.float32),
        mesh=_sc_mesh(),
        scratch_types=[pltpu.VMEM_SHARED((N, D), jnp.float32),
                       pltpu.VMEM((BLK,), jnp.int32),
                       pltpu.VMEM((BLK,), jnp.int32),
                       pltpu.VMEM((BLK,), jnp.float32),
                       pltpu.VMEM((C, D), jnp.float32),
                       pltpu.VMEM((C, D), jnp.float32),
                       pltpu.VMEM((C,), jnp.int32),
                       pltpu.VMEM((C,), jnp.int32),
                       pltpu.VMEM((C,), jnp.int32),
                       pltpu.VMEM((C,), jnp.int32),
                       pltpu.VMEM((WC, D), jnp.float32),
                       pltpu.SemaphoreType.DMA,
                       pltpu.SemaphoreType.DMA],
        name="rgcn_sc_agg",
    )
    return f(hb_flat, dstv, gidx, norm, zwc)


def _dec_body(x2_ref, la_ref, lb_ref, s_ref, ai, bi, ra, rb, ob, sem):
    cid = lax.axis_index("c")
    sid = lax.axis_index("s")
    wid = sid * NC + cid

    def chunk(ci, _):
        base = wid * PP + ci * CD
        pltpu.sync_copy(la_ref.at[pl.ds(base, CD)], ai)
        pltpu.sync_copy(lb_ref.at[pl.ds(base, CD)], bi)

        def g(j, _):
            bi[pl.ds(j * 16, 16)] = bi[pl.ds(j * 16, 16)] + N_M
            return 0
        lax.fori_loop(0, CD // 16, g, 0)
        pltpu.async_copy(x2_ref.at[ai], ra, sem).wait()
        pltpu.async_copy(x2_ref.at[bi], rb, sem).wait()

        def dot_grp(g, _):
            lanes = lax.iota(jnp.int32, 16)
            r16 = jnp.zeros((16,), jnp.float32)
            for jj in range(16):
                row = g * 16 + jj
                acc = ra[row, pl.ds(0, 16)] * rb[row, pl.ds(0, 16)]
                for k in range(1, D // 16):
                    acc = acc + ra[row, pl.ds(k * 16, 16)] * rb[row, pl.ds(k * 16, 16)]
                s_jj = acc[0]
                for l in range(1, 16):
                    s_jj = s_jj + acc[l]
                r16 = jnp.where(lanes == jj, s_jj, r16)
            ob[pl.ds(g * 16, 16)] = r16
            return 0
        lax.fori_loop(0, CD // 16, dot_grp, 0)
        pltpu.sync_copy(ob, s_ref.at[pl.ds(base, CD)])
        return 0
    lax.fori_loop(0, PP // CD, chunk, 0)


def _sc_decoder(x2, la, lb):
    f = pl.kernel(
        _dec_body,
        out_type=jax.ShapeDtypeStruct((B_EDGES,), jnp.float32),
        mesh=_sc_mesh(),
        scratch_types=[pltpu.VMEM((CD,), jnp.int32),
                       pltpu.VMEM((CD,), jnp.int32),
                       pltpu.VMEM((CD, D), jnp.float32),
                       pltpu.VMEM((CD, D), jnp.float32),
                       pltpu.VMEM((CD,), jnp.float32),
                       pltpu.SemaphoreType.DMA],
        name="rgcn_sc_decoder",
    )
    return f(x2, la, lb)


def _wcat(comp, basis, root, bias):
    w = jnp.einsum('rb,bio->rio', comp, basis)
    wc = jnp.concatenate([w.transpose(1, 0, 2).reshape(D, R * D), root], axis=1)
    bc = jnp.concatenate([jnp.zeros((R * D,), jnp.float32), bias]).reshape(1, NCOL)
    return wc, bc


def kernel(sim_m, sim_g, edge_index, edge_type, label_edge,
           lin_m_w, lin_m_b, lin_g_w, lin_g_b,
           comp1, basis1, root1, bias1,
           comp2, basis2, root2, bias2):
    ei = edge_index.astype(jnp.int32)
    et = edge_type.astype(jnp.int32)
    srcv, dstv = ei[0], ei[1]
    le = label_edge.astype(jnp.int32)
    la, lb = le[0], le[1]

    wc1, bc1 = _wcat(comp1, basis1, root1, bias1)
    wc2, bc2 = _wcat(comp2, basis2, root2, bias2)

    xm0 = _matmul(sim_m, lin_m_w, lin_m_b.reshape(1, -1), 1000)
    xg0 = _matmul(sim_g, lin_g_w, lin_g_b.reshape(1, -1), 1000)
    x0 = jnp.concatenate([xm0, xg0], axis=0)

    z8k = jnp.zeros((8000,), jnp.float32)
    onesc = jnp.ones((C,), jnp.float32)
    zwc = jnp.zeros((WC, D), jnp.float32)
    _h0, _h1, norm, gidx = _sc_prep(srcv, dstv, et, z8k, onesc)

    hb1 = _matmul(x0, wc1, bc1, 1000)
    acc1 = _sc_agg(hb1.reshape((R + 1) * N, D), dstv, gidx, norm, zwc)
    hb2 = _fused_matmul(acc1, hb1, wc2, bc2, 1000)
    acc2 = _sc_agg(hb2.reshape((R + 1) * N, D), dstv, gidx, norm, zwc)
    x2 = _add3(acc2, hb2, 1000)
    return _sc_decoder(x2, la, lb)
